# baseline (speedup 1.0000x reference)
import functools

import jax
import jax.numpy as jnp
from jax.experimental import pallas as pl
from jax.experimental.pallas import tpu as pltpu


def _daft_fused_kernel(x_ref, xt_ref, w1t_ref, b1_ref, w2t_ref, b2_ref, o_ref,
                       *, b0):
    C, S = x_ref.shape
    bidx = pl.program_id(0) + b0
    x = x_ref[...]
    pooled = jnp.sum(x, axis=1, keepdims=True) * (1.0 / S)
    lane = jax.lax.broadcasted_iota(jnp.int32, xt_ref.shape, 1)
    xt_col = jnp.sum(jnp.where(lane == bidx, xt_ref[...], 0.0),
                     axis=1, keepdims=True)
    z = jnp.concatenate([pooled, xt_col], axis=0)
    h = jax.lax.dot_general(w1t_ref[...], z, (((1,), (0,)), ((), ())),
                            preferred_element_type=jnp.float32)
    h = jnp.maximum(h + b1_ref[...], 0.0)
    y = jax.lax.dot_general(w2t_ref[...], h, (((1,), (0,)), ((), ())),
                            preferred_element_type=jnp.float32)
    y = y + b2_ref[...]
    o_ref[...] = y[:C, :] * x + y[C:, :]


def _alias_kernel(x_ref, xt_ref, w1t_ref, b1_ref, w2t_ref, b2_ref, prev_ref,
                  o_ref, *, b0):
    _daft_fused_kernel(x_ref, xt_ref, w1t_ref, b1_ref, w2t_ref, b2_ref, o_ref,
                       b0=b0)


def kernel(x_img, x_tab, w1, b1, w2, b2):
    B, C, D, H, W = x_img.shape
    S = D * H * W
    P = x_tab.shape[1]
    hidden = w1.shape[1]
    half = B // 2

    x3 = x_img.reshape(B, C, S)
    xt = x_tab.astype(jnp.float32).T
    w1t = w1.astype(jnp.float32).T
    b1c = b1.astype(jnp.float32).reshape(hidden, 1)
    w2t = w2.astype(jnp.float32).T
    b2c = b2.astype(jnp.float32).reshape(2 * C, 1)

    small_specs = [
        pl.BlockSpec((P, B), lambda b: (0, 0)),
        pl.BlockSpec((hidden, C + P), lambda b: (0, 0)),
        pl.BlockSpec((hidden, 1), lambda b: (0, 0)),
        pl.BlockSpec((2 * C, hidden), lambda b: (0, 0)),
        pl.BlockSpec((2 * C, 1), lambda b: (0, 0)),
    ]
    xspec = pl.BlockSpec((pl.Squeezed(), C, S), lambda b: (b, 0, 0))
    xspec2 = pl.BlockSpec((pl.Squeezed(), C, S), lambda b: (b + half, 0, 0))
    ospec = pl.BlockSpec((pl.Squeezed(), C, S), lambda b: (b, 0, 0))
    ospec2 = pl.BlockSpec((pl.Squeezed(), C, S), lambda b: (b + half, 0, 0))
    cspec = pl.BlockSpec((pl.Squeezed(), C, min(128, S)), lambda b: (0, 0, 0))

    out1 = pl.pallas_call(
        functools.partial(_daft_fused_kernel, b0=0),
        out_shape=jax.ShapeDtypeStruct((B, C, S), x_img.dtype),
        grid=(half,),
        in_specs=[xspec] + small_specs,
        out_specs=ospec,
        compiler_params=pltpu.CompilerParams(
            dimension_semantics=("arbitrary",)),
    )(x3, xt, w1t, b1c, w2t, b2c)

    out = pl.pallas_call(
        functools.partial(_alias_kernel, b0=half),
        out_shape=jax.ShapeDtypeStruct((B, C, S), x_img.dtype),
        grid=(half,),
        in_specs=[xspec2] + small_specs + [cspec],
        out_specs=ospec2,
        input_output_aliases={6: 0},
        compiler_params=pltpu.CompilerParams(
            dimension_semantics=("arbitrary",)),
    )(x3, xt, w1t, b1c, w2t, b2c, out1)

    return out.reshape(B, C, D, H, W)
